# baseline (speedup 1.0000x reference)
import functools

import jax
import jax.numpy as jnp
from jax import lax
from jax.experimental import pallas as pl
from jax.experimental.pallas import tpu as pltpu
from jax.experimental.pallas import tpu_sc as plsc

VOCAB = 100000
DIM = 128
SCALE = float(DIM) ** 0.5

BATCH = 4096
SEQ = 200
TOTAL = BATCH * SEQ

_info = plsc.get_sparse_core_info()
_NC = _info.num_cores
_NS = _info.num_subcores
_NW = _NC * _NS
_BPW = TOTAL // _NW
_CHUNK = 400
_NCHUNK = _BPW // _CHUNK
_NBUF = 2
_NGRP = _NCHUNK // _NBUF

_PREP_ROWS = 10000


def _prep_body(w_ref, o_ref):
    o_ref[...] = w_ref[...] * SCALE

    @pl.when(pl.program_id(0) == 0)
    def _zero_row0():
        o_ref[0:1, :] = jnp.zeros((1, DIM), jnp.float32)


def _prep_table(weights):
    return pl.pallas_call(
        _prep_body,
        grid=(VOCAB // _PREP_ROWS,),
        in_specs=[pl.BlockSpec((_PREP_ROWS, DIM), lambda i: (i, 0))],
        out_specs=pl.BlockSpec((_PREP_ROWS, DIM), lambda i: (i, 0)),
        out_shape=jax.ShapeDtypeStruct((VOCAB, DIM), jnp.float32),
    )(weights)


_mesh = plsc.VectorSubcoreMesh(core_axis_name="c", subcore_axis_name="s")


@functools.partial(
    pl.kernel,
    mesh=_mesh,
    out_type=jax.ShapeDtypeStruct((TOTAL, DIM), jnp.float32),
    scratch_types=[
        pltpu.VMEM((_BPW,), jnp.int32),
        pltpu.VMEM((_NBUF, _CHUNK, DIM), jnp.float32),
        pltpu.SemaphoreType.DMA((_NBUF,)),
        pltpu.SemaphoreType.DMA((_NBUF,)),
    ],
)
def _sc_gather(table_hbm, idx_hbm, out_hbm, idx_v, rows_v, gsem, ssem):
    wid = lax.axis_index("s") * _NC + lax.axis_index("c")
    base = wid * _BPW

    pltpu.sync_copy(idx_hbm.at[pl.ds(base, _BPW)], idx_v)

    def start_gather(c, b):
        idx = idx_v.at[pl.ds(c * _CHUNK, _CHUNK)]
        pltpu.async_copy(table_hbm.at[idx], rows_v.at[b], gsem.at[b])

    def wait_gather(c, b):
        idx = idx_v.at[pl.ds(c * _CHUNK, _CHUNK)]
        pltpu.make_async_copy(table_hbm.at[idx], rows_v.at[b], gsem.at[b]).wait()

    def start_scatter(c, b):
        dst = out_hbm.at[pl.ds(base + c * _CHUNK, _CHUNK)]
        pltpu.async_copy(rows_v.at[b], dst, ssem.at[b])

    def wait_scatter(c, b):
        dst = out_hbm.at[pl.ds(base + c * _CHUNK, _CHUNK)]
        pltpu.make_async_copy(rows_v.at[b], dst, ssem.at[b]).wait()

    start_gather(0, 0)
    for b in range(1, _NBUF):
        start_gather(b, b)
        wait_gather(b - 1, b - 1)
        start_scatter(b - 1, b - 1)

    def body(g, carry):
        for b in range(_NBUF):
            c = g * _NBUF + b
            wait_scatter(c - _NBUF, b)
            start_gather(c, b)
            wait_gather(c - 1, (b - 1) % _NBUF)
            start_scatter(c - 1, (b - 1) % _NBUF)
        return carry

    lax.fori_loop(1, _NGRP, body, 0)

    last = _NCHUNK - 1
    wait_gather(last, _NBUF - 1)
    start_scatter(last, _NBUF - 1)
    for b in range(_NBUF):
        wait_scatter(last - (_NBUF - 1) + b, b)


def kernel(inputs, shared_weights):
    scaled = _prep_table(shared_weights)
    flat_idx = inputs.reshape(TOTAL).astype(jnp.int32)
    out = _sc_gather(scaled, flat_idx)
    return out.reshape(BATCH, SEQ, DIM)


# speedup vs baseline: 1.0051x; 1.0051x over previous
import functools

import jax
import jax.numpy as jnp
from jax import lax
from jax.experimental import pallas as pl
from jax.experimental.pallas import tpu as pltpu
from jax.experimental.pallas import tpu_sc as plsc

VOCAB = 100000
DIM = 128
SCALE = float(DIM) ** 0.5

BATCH = 4096
SEQ = 200
TOTAL = BATCH * SEQ

_info = plsc.get_sparse_core_info()
_NC = _info.num_cores
_NS = _info.num_subcores
_NW = _NC * _NS
_BPW = TOTAL // _NW
_CHUNK = 400
_NCHUNK = _BPW // _CHUNK
_NBUF = 2
_NGRP = _NCHUNK // _NBUF

_PREP_ROWS = 20000


def _prep_body(w_ref, o_ref):
    o_ref[...] = w_ref[...] * SCALE

    @pl.when(pl.program_id(0) == 0)
    def _zero_row0():
        o_ref[0:1, :] = jnp.zeros((1, DIM), jnp.float32)


def _prep_table(weights):
    return pl.pallas_call(
        _prep_body,
        grid=(VOCAB // _PREP_ROWS,),
        in_specs=[pl.BlockSpec((_PREP_ROWS, DIM), lambda i: (i, 0))],
        out_specs=pl.BlockSpec((_PREP_ROWS, DIM), lambda i: (i, 0)),
        out_shape=jax.ShapeDtypeStruct((VOCAB, DIM), jnp.float32),
    )(weights)


_mesh = plsc.VectorSubcoreMesh(core_axis_name="c", subcore_axis_name="s")


@functools.partial(
    pl.kernel,
    mesh=_mesh,
    out_type=jax.ShapeDtypeStruct((TOTAL, DIM), jnp.float32),
    scratch_types=[
        pltpu.VMEM((_BPW,), jnp.int32),
        pltpu.VMEM((_NBUF, _CHUNK, DIM), jnp.float32),
        pltpu.SemaphoreType.DMA((_NBUF,)),
        pltpu.SemaphoreType.DMA((_NBUF,)),
    ],
)
def _sc_gather(table_hbm, idx_hbm, out_hbm, idx_v, rows_v, gsem, ssem):
    wid = lax.axis_index("s") * _NC + lax.axis_index("c")
    base = wid * _BPW

    pltpu.sync_copy(idx_hbm.at[pl.ds(base, _BPW)], idx_v)

    def start_gather(c, b):
        idx = idx_v.at[pl.ds(c * _CHUNK, _CHUNK)]
        pltpu.async_copy(table_hbm.at[idx], rows_v.at[b], gsem.at[b])

    def wait_gather(c, b):
        idx = idx_v.at[pl.ds(c * _CHUNK, _CHUNK)]
        pltpu.make_async_copy(table_hbm.at[idx], rows_v.at[b], gsem.at[b]).wait()

    def start_scatter(c, b):
        dst = out_hbm.at[pl.ds(base + c * _CHUNK, _CHUNK)]
        pltpu.async_copy(rows_v.at[b], dst, ssem.at[b])

    def wait_scatter(c, b):
        dst = out_hbm.at[pl.ds(base + c * _CHUNK, _CHUNK)]
        pltpu.make_async_copy(rows_v.at[b], dst, ssem.at[b]).wait()

    start_gather(0, 0)
    for b in range(1, _NBUF):
        start_gather(b, b)
        wait_gather(b - 1, b - 1)
        start_scatter(b - 1, b - 1)

    def body(g, carry):
        for b in range(_NBUF):
            c = g * _NBUF + b
            wait_scatter(c - _NBUF, b)
            start_gather(c, b)
            wait_gather(c - 1, (b - 1) % _NBUF)
            start_scatter(c - 1, (b - 1) % _NBUF)
        return carry

    lax.fori_loop(1, _NGRP, body, 0)

    last = _NCHUNK - 1
    wait_gather(last, _NBUF - 1)
    start_scatter(last, _NBUF - 1)
    for b in range(_NBUF):
        wait_scatter(last - (_NBUF - 1) + b, b)


def kernel(inputs, shared_weights):
    scaled = _prep_table(shared_weights)
    flat_idx = inputs.reshape(TOTAL).astype(jnp.int32)
    out = _sc_gather(scaled, flat_idx)
    return out.reshape(BATCH, SEQ, DIM)


# speedup vs baseline: 1.0064x; 1.0013x over previous
import functools

import jax
import jax.numpy as jnp
from jax import lax
from jax.experimental import pallas as pl
from jax.experimental.pallas import tpu as pltpu
from jax.experimental.pallas import tpu_sc as plsc

VOCAB = 100000
DIM = 128
SCALE = float(DIM) ** 0.5

BATCH = 4096
SEQ = 200
TOTAL = BATCH * SEQ

_info = plsc.get_sparse_core_info()
_NC = _info.num_cores
_NS = _info.num_subcores
_NW = _NC * _NS
_BPW = TOTAL // _NW
_CHUNK = 400
_NCHUNK = _BPW // _CHUNK
_NBUF = 2
_NGRP = _NCHUNK // _NBUF

_PREP_ROWS = 25000


def _prep_body(w_ref, o_ref):
    o_ref[...] = w_ref[...] * SCALE

    @pl.when(pl.program_id(0) == 0)
    def _zero_row0():
        o_ref[0:1, :] = jnp.zeros((1, DIM), jnp.float32)


def _prep_table(weights):
    return pl.pallas_call(
        _prep_body,
        grid=(VOCAB // _PREP_ROWS,),
        in_specs=[pl.BlockSpec((_PREP_ROWS, DIM), lambda i: (i, 0))],
        out_specs=pl.BlockSpec((_PREP_ROWS, DIM), lambda i: (i, 0)),
        out_shape=jax.ShapeDtypeStruct((VOCAB, DIM), jnp.float32),
    )(weights)


_mesh = plsc.VectorSubcoreMesh(core_axis_name="c", subcore_axis_name="s")


@functools.partial(
    pl.kernel,
    mesh=_mesh,
    out_type=jax.ShapeDtypeStruct((TOTAL, DIM), jnp.float32),
    scratch_types=[
        pltpu.VMEM((_BPW,), jnp.int32),
        pltpu.VMEM((_NBUF, _CHUNK, DIM), jnp.float32),
        pltpu.SemaphoreType.DMA((_NBUF,)),
        pltpu.SemaphoreType.DMA((_NBUF,)),
    ],
)
def _sc_gather(table_hbm, idx_hbm, out_hbm, idx_v, rows_v, gsem, ssem):
    wid = lax.axis_index("s") * _NC + lax.axis_index("c")
    base = wid * _BPW

    pltpu.sync_copy(idx_hbm.at[pl.ds(base, _BPW)], idx_v)

    def start_gather(c, b):
        idx = idx_v.at[pl.ds(c * _CHUNK, _CHUNK)]
        pltpu.async_copy(table_hbm.at[idx], rows_v.at[b], gsem.at[b])

    def wait_gather(c, b):
        idx = idx_v.at[pl.ds(c * _CHUNK, _CHUNK)]
        pltpu.make_async_copy(table_hbm.at[idx], rows_v.at[b], gsem.at[b]).wait()

    def start_scatter(c, b):
        dst = out_hbm.at[pl.ds(base + c * _CHUNK, _CHUNK)]
        pltpu.async_copy(rows_v.at[b], dst, ssem.at[b])

    def wait_scatter(c, b):
        dst = out_hbm.at[pl.ds(base + c * _CHUNK, _CHUNK)]
        pltpu.make_async_copy(rows_v.at[b], dst, ssem.at[b]).wait()

    start_gather(0, 0)
    for b in range(1, _NBUF):
        start_gather(b, b)
        wait_gather(b - 1, b - 1)
        start_scatter(b - 1, b - 1)

    def body(g, carry):
        for b in range(_NBUF):
            c = g * _NBUF + b
            wait_scatter(c - _NBUF, b)
            start_gather(c, b)
            wait_gather(c - 1, (b - 1) % _NBUF)
            start_scatter(c - 1, (b - 1) % _NBUF)
        return carry

    lax.fori_loop(1, _NGRP, body, 0)

    last = _NCHUNK - 1
    wait_gather(last, _NBUF - 1)
    start_scatter(last, _NBUF - 1)
    for b in range(_NBUF):
        wait_scatter(last - (_NBUF - 1) + b, b)


def kernel(inputs, shared_weights):
    scaled = _prep_table(shared_weights)
    flat_idx = inputs.reshape(TOTAL).astype(jnp.int32)
    out = _sc_gather(scaled, flat_idx)
    return out.reshape(BATCH, SEQ, DIM)
